# baseline (speedup 1.0000x reference)
import functools

import jax
import jax.numpy as jnp
from jax import lax
from jax.experimental import pallas as pl
from jax.experimental.pallas import tpu as pltpu
from jax.experimental.pallas import tpu_sc as plsc

VOCAB = 1_000_000
DIM = 64
BATCH = 16384
NWORDS = 20

NC = 2
NS = 16
NWK = NC * NS
BPW = BATCH // NWK
CB = 32
NCHUNK = BPW // CB
IDX_SPLIT = 128
NSPLIT = (CB * NWORDS) // IDX_SPLIT


def _w2v_body(cen_words, ctx_words, cen_table, ctx_table, out,
              cidx_v, xidx_v, cen_v, ctx_v, sco_v, sem):
    wid = lax.axis_index("s") * NC + lax.axis_index("c")
    base = wid * BPW
    iot = lax.iota(jnp.int32, 16)

    def chunk(k, carry):
        cb = base + k * CB
        pltpu.sync_copy(cen_words.at[pl.ds(cb, CB)], cidx_v)
        pltpu.sync_copy(ctx_words.at[pl.ds(cb * NWORDS, CB * NWORDS)], xidx_v)
        copies = [pltpu.async_copy(cen_table.at[cidx_v], cen_v, sem)]
        for j in range(NSPLIT):
            copies.append(pltpu.async_copy(
                ctx_table.at[xidx_v.at[pl.ds(j * IDX_SPLIT, IDX_SPLIT)]],
                ctx_v.at[pl.ds(j * IDX_SPLIT, IDX_SPLIT)], sem))
        for c in copies:
            c.wait()
        def unit(u, carry2):
            b0 = u * 4
            accs = [jnp.zeros((16,), jnp.float32) for _ in range(5)]
            for i in range(4):
                b = b0 + i
                cvs = [cen_v[b, pl.ds(j * 16, 16)] for j in range(DIM // 16)]
                for n in range(NWORDS):
                    row = b * NWORDS + n
                    p = ctx_v[row, pl.ds(0, 16)] * cvs[0]
                    for j in range(1, DIM // 16):
                        p = p + ctx_v[row, pl.ds(j * 16, 16)] * cvs[j]
                    s = jnp.sum(p)
                    fp = i * NWORDS + n
                    accs[fp // 16] = jnp.where(
                        iot == (fp % 16), jnp.full((16,), s, jnp.float32),
                        accs[fp // 16])
            for g in range(5):
                sco_v[pl.ds(u * 80 + g * 16, 16)] = accs[g]
            return carry2

        lax.fori_loop(0, CB // 4, unit, 0)
        pltpu.sync_copy(sco_v, out.at[pl.ds(cb * NWORDS, CB * NWORDS)])
        return carry

    lax.fori_loop(0, NCHUNK, chunk, 0)


_w2v = functools.partial(
    pl.kernel,
    mesh=plsc.VectorSubcoreMesh(core_axis_name="c", subcore_axis_name="s"),
    compiler_params=pltpu.CompilerParams(
        needs_layout_passes=False, use_tc_tiling_on_sc=False),
    out_type=jax.ShapeDtypeStruct((BATCH * NWORDS,), jnp.float32),
    scratch_types=[
        pltpu.VMEM((CB,), jnp.int32),
        pltpu.VMEM((CB * NWORDS,), jnp.int32),
        pltpu.VMEM((CB, DIM), jnp.float32),
        pltpu.VMEM((CB * NWORDS, DIM), jnp.float32),
        pltpu.VMEM((CB * NWORDS,), jnp.float32),
        pltpu.SemaphoreType.DMA,
    ],
)(_w2v_body)


@jax.jit
def kernel(center_words, context_words, center_table, context_table):
    ctx_flat = context_words.astype(jnp.int32).reshape(BATCH * NWORDS)
    flat = _w2v(center_words.astype(jnp.int32), ctx_flat,
                center_table, context_table)
    return flat.reshape(BATCH, NWORDS)
